# baseline (speedup 1.0000x reference)
import jax
import jax.numpy as jnp
from jax.experimental import pallas as pl


def _copy_body(w_ref, o_ref):
    o_ref[...] = w_ref[...][None]


def kernel(tokens, W_pos):
    batch, seq = tokens.shape
    d = W_pos.shape[-1]
    pos = W_pos[:seq]
    BS = 512
    grid = (seq // BS, batch)
    return pl.pallas_call(
        _copy_body,
        grid=grid,
        in_specs=[pl.BlockSpec((BS, d), lambda i, b: (i, 0))],
        out_specs=pl.BlockSpec((1, BS, d), lambda i, b: (b, i, 0)),
        out_shape=jax.ShapeDtypeStruct((batch, seq, d), W_pos.dtype),
    )(pos)


# speedup vs baseline: 1.5494x; 1.5494x over previous
import functools

import jax
import jax.numpy as jnp
from jax.experimental import pallas as pl
from jax.experimental.pallas import tpu as pltpu

_CHUNK = 1024
_NBUF = 3


def _dma_body(batch, seq, d, chunk, nbuf, w_hbm, o_hbm, vmem, in_sems, out_sems):
    n = seq // chunk

    def read(i):
        buf = i % nbuf
        return pltpu.make_async_copy(
            w_hbm.at[pl.ds(i * chunk, chunk)], vmem.at[buf], in_sems.at[buf])

    def write(i, b):
        buf = i % nbuf
        return pltpu.make_async_copy(
            vmem.at[buf], o_hbm.at[b, pl.ds(i * chunk, chunk)], out_sems.at[buf])

    for i in range(min(nbuf - 1, n)):
        read(i).start()
    for i in range(n):
        read(i).wait()
        for b in range(batch):
            write(i, b).start()
        j = i + nbuf - 1
        if j < n:
            if j - nbuf >= 0:
                for b in range(batch):
                    write(j - nbuf, b).wait()
            read(j).start()
    for i in range(max(0, n - nbuf), n):
        for b in range(batch):
            write(i, b).wait()


def kernel(tokens, W_pos):
    batch, seq = tokens.shape
    d = W_pos.shape[-1]
    pos = W_pos[:seq]
    chunk = min(_CHUNK, seq)
    return pl.pallas_call(
        functools.partial(_dma_body, batch, seq, d, chunk, _NBUF),
        in_specs=[pl.BlockSpec(memory_space=pl.ANY)],
        out_specs=pl.BlockSpec(memory_space=pl.ANY),
        out_shape=jax.ShapeDtypeStruct((batch, seq, d), W_pos.dtype),
        scratch_shapes=[
            pltpu.VMEM((_NBUF, chunk, d), W_pos.dtype),
            pltpu.SemaphoreType.DMA((_NBUF,)),
            pltpu.SemaphoreType.DMA((_NBUF,)),
        ],
    )(pos)


# speedup vs baseline: 1.5735x; 1.0156x over previous
import functools

import jax
import jax.numpy as jnp
from jax.experimental import pallas as pl
from jax.experimental.pallas import tpu as pltpu

_CHUNK = 512


def _dma_body(batch, seq, d, chunk, w_hbm, o_hbm, vmem, in_sems, out_sems):
    n = seq // chunk

    def read(i):
        return pltpu.make_async_copy(
            w_hbm.at[pl.ds(i * chunk, chunk)],
            vmem.at[pl.ds(i * chunk, chunk)], in_sems.at[i])

    def write(i, b):
        return pltpu.make_async_copy(
            vmem.at[pl.ds(i * chunk, chunk)],
            o_hbm.at[b, pl.ds(i * chunk, chunk)], out_sems.at[i])

    for i in range(n):
        read(i).start()
    for i in range(n):
        read(i).wait()
        for b in range(batch):
            write(i, b).start()
    for i in range(n):
        for b in range(batch):
            write(i, b).wait()


def kernel(tokens, W_pos):
    batch, seq = tokens.shape
    d = W_pos.shape[-1]
    pos = W_pos[:seq]
    chunk = min(_CHUNK, seq)
    n = seq // chunk
    return pl.pallas_call(
        functools.partial(_dma_body, batch, seq, d, chunk),
        in_specs=[pl.BlockSpec(memory_space=pl.ANY)],
        out_specs=pl.BlockSpec(memory_space=pl.ANY),
        out_shape=jax.ShapeDtypeStruct((batch, seq, d), W_pos.dtype),
        scratch_shapes=[
            pltpu.VMEM((seq, d), W_pos.dtype),
            pltpu.SemaphoreType.DMA((n,)),
            pltpu.SemaphoreType.DMA((n,)),
        ],
    )(pos)
